# Initial kernel scaffold; baseline (speedup 1.0000x reference)
import jax
import jax.numpy as jnp
from jax.experimental import pallas as pl


def kernel(idx, wte, wpe):
    raise NotImplementedError("write your pallas kernel here")


# speedup vs baseline: 1.3395x; 1.3395x over previous
import functools

import jax
import jax.numpy as jnp
from jax import lax
from jax.experimental import pallas as pl
from jax.experimental.pallas import tpu as pltpu
from jax.experimental.pallas import tpu_sc as plsc

IDX_CHUNK = 128


@functools.lru_cache(maxsize=None)
def _build(flat, V, S, D):
    info = plsc.get_sparse_core_info()
    NC, NS = info.num_cores, info.num_subcores
    NW = NC * NS
    b_per_w = flat // NW
    n_chunks = b_per_w // IDX_CHUNK
    assert flat % NW == 0 and b_per_w % IDX_CHUNK == 0
    assert S % b_per_w == 0
    chunks_per_s = S // IDX_CHUNK

    mesh = plsc.VectorSubcoreMesh(core_axis_name="c", subcore_axis_name="s")

    @functools.partial(
        pl.kernel,
        mesh=mesh,
        out_type=jax.ShapeDtypeStruct((flat, D), jnp.float32),
        scratch_types=[
            pltpu.VMEM((n_chunks, IDX_CHUNK), jnp.int32),
            pltpu.VMEM((b_per_w, D), jnp.float32),
            pltpu.SemaphoreType.DMA,
        ],
    )
    def k(idx_hbm, wte_hbm, wpe_hbm, out_hbm, idx_v, rows_v, sem):
        wid = lax.axis_index("s") * NC + lax.axis_index("c")
        base = wid * b_per_w
        s0 = lax.rem(wid, S // b_per_w) * b_per_w
        pltpu.sync_copy(idx_hbm.at[pl.ds(wid * n_chunks, n_chunks)], idx_v)
        pltpu.sync_copy(wpe_hbm.at[pl.ds(s0, b_per_w)], rows_v)
        copies = []
        for j in range(n_chunks):
            copies.append(
                pltpu.async_copy(
                    wte_hbm.at[idx_v.at[j]],
                    rows_v.at[pl.ds(j * IDX_CHUNK, IDX_CHUNK)],
                    sem,
                    add=True,
                )
            )
        for c in copies:
            c.wait()
        pltpu.sync_copy(rows_v, out_hbm.at[pl.ds(base, b_per_w)])

    del chunks_per_s
    return k


def kernel(idx, wte, wpe):
    B, S = idx.shape
    V, D = wte.shape
    flat = B * S
    idx_flat = idx.reshape(flat // IDX_CHUNK, IDX_CHUNK).astype(jnp.int32)
    out = _build(flat, V, S, D)(idx_flat, wte, wpe)
    return out.reshape(B, S, D)


# speedup vs baseline: 1.3848x; 1.0339x over previous
import functools

import jax
import jax.numpy as jnp
from jax import lax
from jax.experimental import pallas as pl
from jax.experimental.pallas import tpu as pltpu
from jax.experimental.pallas import tpu_sc as plsc

IDX_CHUNK = 128


@functools.lru_cache(maxsize=None)
def _build(B, S, V, D):
    info = plsc.get_sparse_core_info()
    NC, NS = info.num_cores, info.num_subcores
    NW = NC * NS
    flat = B * S
    b_per_w = flat // NW
    n_chunks = b_per_w // IDX_CHUNK
    assert flat % NW == 0 and b_per_w % IDX_CHUNK == 0
    assert S % b_per_w == 0
    w_per_b = S // b_per_w

    mesh = plsc.VectorSubcoreMesh(core_axis_name="c", subcore_axis_name="s")

    @functools.partial(
        pl.kernel,
        mesh=mesh,
        out_type=jax.ShapeDtypeStruct((B, S, D), jnp.float32),
        scratch_types=[
            pltpu.VMEM((n_chunks, IDX_CHUNK), jnp.int32),
            pltpu.VMEM((b_per_w, D), jnp.float32),
            [pltpu.SemaphoreType.DMA] * n_chunks,
            [pltpu.SemaphoreType.DMA] * n_chunks,
            pltpu.SemaphoreType.DMA,
        ],
    )
    def k(idx_hbm, wte_hbm, wpe_hbm, out_hbm, idx_v, rows_v, sem_w, sem_g,
          sem_o):
        wid = lax.axis_index("s") * NC + lax.axis_index("c")
        b = lax.div(wid, w_per_b)
        s0 = lax.rem(wid, w_per_b) * b_per_w
        wpe_cp = []
        for j in range(n_chunks):
            wpe_cp.append(
                pltpu.async_copy(
                    wpe_hbm.at[pl.ds(s0 + j * IDX_CHUNK, IDX_CHUNK)],
                    rows_v.at[pl.ds(j * IDX_CHUNK, IDX_CHUNK)],
                    sem_w[j],
                )
            )
        for j in range(n_chunks):
            pltpu.sync_copy(
                idx_hbm.at[b, pl.ds(s0 + j * IDX_CHUNK, IDX_CHUNK)],
                idx_v.at[j],
            )
        g_cp = []
        for j in range(n_chunks):
            wpe_cp[j].wait()
            g_cp.append(
                pltpu.async_copy(
                    wte_hbm.at[idx_v.at[j]],
                    rows_v.at[pl.ds(j * IDX_CHUNK, IDX_CHUNK)],
                    sem_g[j],
                    add=True,
                )
            )
        o_cp = []
        for j in range(n_chunks):
            g_cp[j].wait()
            o_cp.append(
                pltpu.async_copy(
                    rows_v.at[pl.ds(j * IDX_CHUNK, IDX_CHUNK)],
                    out_hbm.at[b, pl.ds(s0 + j * IDX_CHUNK, IDX_CHUNK)],
                    sem_o,
                )
            )
        for c in o_cp:
            c.wait()

    return k


def kernel(idx, wte, wpe):
    B, S = idx.shape
    V, D = wte.shape
    return _build(B, S, V, D)(idx.astype(jnp.int32), wte, wpe)
